# baseline (speedup 1.0000x reference)
import functools

import jax
import jax.numpy as jnp
from jax import lax
from jax.experimental import pallas as pl
from jax.experimental.pallas import tpu as pltpu
from jax.experimental.pallas import tpu_sc as plsc

_info = plsc.get_sparse_core_info()
_NC, _NS = _info.num_cores, _info.num_subcores
_NW = _NC * _NS

_CHUNK = 100
_K = 8


def _make_gather(V, D, N):
    assert N % (_NW * _CHUNK) == 0
    n_rows = N // _CHUNK
    n_chunks = n_rows // _NW
    assert n_chunks % (2 * _K) == 0
    n_groups = n_chunks // _K
    mesh = plsc.VectorSubcoreMesh(core_axis_name="c", subcore_axis_name="s")

    scratch = [pltpu.VMEM((n_chunks, _CHUNK), jnp.int32)]
    scratch += [pltpu.VMEM((_CHUNK, D), jnp.float32) for _ in range(2 * _K)]
    scratch += [pltpu.SemaphoreType.DMA for _ in range(4)]

    @functools.partial(
        pl.kernel,
        mesh=mesh,
        out_type=jax.ShapeDtypeStruct((n_rows, _CHUNK, D), jnp.float32),
        scratch_types=scratch,
        compiler_params=pltpu.CompilerParams(use_tc_tiling_on_sc=False),
    )
    def gather_kernel(table_hbm, idx_hbm, out_hbm, idx_v, *rest):
        rows = rest[: 2 * _K]
        gsems = rest[2 * _K : 2 * _K + 2]
        wsems = rest[2 * _K + 2 :]
        wid = lax.axis_index("s") * _NC + lax.axis_index("c")
        base = wid * n_chunks

        pltpu.sync_copy(idx_hbm.at[wid], idx_v)

        def start_gathers(g, h):
            for b in range(_K):
                j = g * _K + b
                pltpu.async_copy(table_hbm.at[idx_v.at[j]], rows[h * _K + b], gsems[h])

        def drain_gathers(h):
            for b in range(_K):
                pltpu.make_async_copy(
                    table_hbm.at[idx_v.at[0]], rows[h * _K + b], gsems[h]
                ).wait()

        def start_writebacks(g, h):
            for b in range(_K):
                j = g * _K + b
                pltpu.async_copy(rows[h * _K + b], out_hbm.at[base + j], wsems[h])

        def drain_writebacks(h):
            for b in range(_K):
                pltpu.make_async_copy(
                    rows[h * _K + b], out_hbm.at[0], wsems[h]
                ).wait()

        def half_step(g, h, o):
            drain_gathers(h)

            @pl.when(g + 1 < n_groups)
            def _():
                @pl.when(g >= 1)
                def _():
                    drain_writebacks(o)

                start_gathers(g + 1, o)

            start_writebacks(g, h)

        start_gathers(0, 0)

        def body(g, carry):
            half_step(g, 0, 1)
            half_step(g + 1, 1, 0)
            return carry

        lax.fori_loop(0, n_groups // 2, lambda i, c: body(i * 2, c), 0)
        drain_writebacks(1)

    return gather_kernel


def _seq_lens_body(x_ref, o_ref):
    o_ref[...] = jnp.sum((x_ref[...] != 0).astype(jnp.int32), axis=1)


def kernel(x, table):
    B_, L_ = x.shape
    V, D = table.shape
    N = B_ * L_
    x_i32 = x.astype(jnp.int32)
    idx3 = x_i32.reshape(_NW, N // (_NW * _CHUNK), _CHUNK)

    emb3 = _make_gather(V, D, N)(table, idx3)
    emb = emb3.reshape(B_, L_, D)

    seq_lens = pl.pallas_call(
        _seq_lens_body,
        out_shape=jax.ShapeDtypeStruct((B_,), jnp.int32),
    )(x_i32)

    return (emb, seq_lens)


# speedup vs baseline: 1.2742x; 1.2742x over previous
import functools

import jax
import jax.numpy as jnp
from jax import lax
from jax.experimental import pallas as pl
from jax.experimental.pallas import tpu as pltpu
from jax.experimental.pallas import tpu_sc as plsc

_info = plsc.get_sparse_core_info()
_NC, _NS = _info.num_cores, _info.num_subcores
_NW = _NC * _NS

_CHUNK = 128
_K = 5


def _make_gather(V, D, N):
    assert N % (_NW * _CHUNK) == 0
    n_rows = N // _CHUNK
    n_chunks = n_rows // _NW
    assert n_chunks % (2 * _K) == 0
    n_groups = n_chunks // _K
    mesh = plsc.VectorSubcoreMesh(core_axis_name="c", subcore_axis_name="s")

    scratch = [pltpu.VMEM((n_chunks, _CHUNK), jnp.int32)]
    scratch += [pltpu.VMEM((_CHUNK, D), jnp.float32) for _ in range(2 * _K)]
    scratch += [pltpu.SemaphoreType.DMA for _ in range(4)]

    @functools.partial(
        pl.kernel,
        mesh=mesh,
        out_type=jax.ShapeDtypeStruct((n_rows, _CHUNK, D), jnp.float32),
        scratch_types=scratch,
        compiler_params=pltpu.CompilerParams(use_tc_tiling_on_sc=False),
    )
    def gather_kernel(table_hbm, idx_hbm, out_hbm, idx_v, *rest):
        rows = rest[: 2 * _K]
        gsems = rest[2 * _K : 2 * _K + 2]
        wsems = rest[2 * _K + 2 :]
        wid = lax.axis_index("s") * _NC + lax.axis_index("c")
        base = wid * n_chunks

        pltpu.sync_copy(idx_hbm.at[wid], idx_v)

        def start_gathers(g, h):
            for b in range(_K):
                j = g * _K + b
                pltpu.async_copy(table_hbm.at[idx_v.at[j]], rows[h * _K + b], gsems[h])

        def drain_gathers(h):
            for b in range(_K):
                pltpu.make_async_copy(
                    table_hbm.at[idx_v.at[0]], rows[h * _K + b], gsems[h]
                ).wait()

        def start_writebacks(g, h):
            for b in range(_K):
                j = g * _K + b
                pltpu.async_copy(rows[h * _K + b], out_hbm.at[base + j], wsems[h])

        def drain_writebacks(h):
            for b in range(_K):
                pltpu.make_async_copy(
                    rows[h * _K + b], out_hbm.at[0], wsems[h]
                ).wait()

        def half_step(g, h, o):
            drain_gathers(h)

            @pl.when(g + 1 < n_groups)
            def _():
                @pl.when(g >= 1)
                def _():
                    drain_writebacks(o)

                start_gathers(g + 1, o)

            start_writebacks(g, h)

        start_gathers(0, 0)

        def body(g, carry):
            half_step(g, 0, 1)
            half_step(g + 1, 1, 0)
            return carry

        lax.fori_loop(0, n_groups // 2, lambda i, c: body(i * 2, c), 0)
        drain_writebacks(1)

    return gather_kernel


def _seq_lens_body(x_ref, o_ref):
    o_ref[...] = jnp.sum((x_ref[...] != 0).astype(jnp.int32), axis=1)


def kernel(x, table):
    B_, L_ = x.shape
    V, D = table.shape
    N = B_ * L_
    x_i32 = x.astype(jnp.int32)
    idx3 = x_i32.reshape(_NW, N // (_NW * _CHUNK), _CHUNK)

    emb3 = _make_gather(V, D, N)(table, idx3)
    emb = emb3.reshape(B_, L_, D)

    seq_lens = pl.pallas_call(
        _seq_lens_body,
        out_shape=jax.ShapeDtypeStruct((B_,), jnp.int32),
    )(x_i32)

    return (emb, seq_lens)


# speedup vs baseline: 1.2871x; 1.0101x over previous
import functools

import jax
import jax.numpy as jnp
from jax import lax
from jax.experimental import pallas as pl
from jax.experimental.pallas import tpu as pltpu
from jax.experimental.pallas import tpu_sc as plsc

_info = plsc.get_sparse_core_info()
_NC, _NS = _info.num_cores, _info.num_subcores
_NW = _NC * _NS

_CHUNK = 128
_K = 10


def _make_gather(V, D, N):
    assert N % (_NW * _CHUNK) == 0
    n_rows = N // _CHUNK
    n_chunks = n_rows // _NW
    assert n_chunks % (2 * _K) == 0
    n_groups = n_chunks // _K
    mesh = plsc.VectorSubcoreMesh(core_axis_name="c", subcore_axis_name="s")

    scratch = [pltpu.VMEM((n_chunks, _CHUNK), jnp.int32)]
    scratch += [pltpu.VMEM((_CHUNK, D), jnp.float32) for _ in range(2 * _K)]
    scratch += [pltpu.SemaphoreType.DMA for _ in range(4)]

    @functools.partial(
        pl.kernel,
        mesh=mesh,
        out_type=jax.ShapeDtypeStruct((n_rows, _CHUNK, D), jnp.float32),
        scratch_types=scratch,
        compiler_params=pltpu.CompilerParams(use_tc_tiling_on_sc=False),
    )
    def gather_kernel(table_hbm, idx_hbm, out_hbm, idx_v, *rest):
        rows = rest[: 2 * _K]
        gsems = rest[2 * _K : 2 * _K + 2]
        wsems = rest[2 * _K + 2 :]
        wid = lax.axis_index("s") * _NC + lax.axis_index("c")
        base = wid * n_chunks

        pltpu.sync_copy(idx_hbm.at[wid], idx_v)

        def start_gathers(g, h):
            for b in range(_K):
                j = g * _K + b
                pltpu.async_copy(table_hbm.at[idx_v.at[j]], rows[h * _K + b], gsems[h])

        def drain_gathers(h):
            for b in range(_K):
                pltpu.make_async_copy(
                    table_hbm.at[idx_v.at[0]], rows[h * _K + b], gsems[h]
                ).wait()

        def start_writebacks(g, h):
            for b in range(_K):
                j = g * _K + b
                pltpu.async_copy(rows[h * _K + b], out_hbm.at[base + j], wsems[h])

        def drain_writebacks(h):
            for b in range(_K):
                pltpu.make_async_copy(
                    rows[h * _K + b], out_hbm.at[0], wsems[h]
                ).wait()

        def half_step(g, h, o):
            drain_gathers(h)

            @pl.when(g + 1 < n_groups)
            def _():
                @pl.when(g >= 1)
                def _():
                    drain_writebacks(o)

                start_gathers(g + 1, o)

            start_writebacks(g, h)

        start_gathers(0, 0)

        def body(g, carry):
            half_step(g, 0, 1)
            half_step(g + 1, 1, 0)
            return carry

        lax.fori_loop(0, n_groups // 2, lambda i, c: body(i * 2, c), 0)
        drain_writebacks(1)

    return gather_kernel


def _seq_lens_body(x_ref, o_ref):
    o_ref[...] = jnp.sum((x_ref[...] != 0).astype(jnp.int32), axis=1)


def kernel(x, table):
    B_, L_ = x.shape
    V, D = table.shape
    N = B_ * L_
    x_i32 = x.astype(jnp.int32)
    idx3 = x_i32.reshape(_NW, N // (_NW * _CHUNK), _CHUNK)

    emb3 = _make_gather(V, D, N)(table, idx3)
    emb = emb3.reshape(B_, L_, D)

    seq_lens = pl.pallas_call(
        _seq_lens_body,
        out_shape=jax.ShapeDtypeStruct((B_,), jnp.int32),
    )(x_i32)

    return (emb, seq_lens)


# speedup vs baseline: 1.2891x; 1.0016x over previous
import functools

import jax
import jax.numpy as jnp
from jax import lax
from jax.experimental import pallas as pl
from jax.experimental.pallas import tpu as pltpu
from jax.experimental.pallas import tpu_sc as plsc

_info = plsc.get_sparse_core_info()
_NC, _NS = _info.num_cores, _info.num_subcores
_NW = _NC * _NS

_CHUNK = 128
_K = 10


def _make_gather(V, D, N):
    assert N % (_NW * _CHUNK) == 0
    n_rows = N // _CHUNK
    n_chunks = n_rows // _NW
    assert n_chunks % (2 * _K) == 0
    n_groups = n_chunks // _K
    mesh = plsc.VectorSubcoreMesh(core_axis_name="c", subcore_axis_name="s")

    scratch = [pltpu.VMEM((n_chunks, _CHUNK), jnp.int32)]
    scratch += [pltpu.VMEM((_K * _CHUNK, D), jnp.float32) for _ in range(2)]
    scratch += [pltpu.SemaphoreType.DMA for _ in range(4)]

    @functools.partial(
        pl.kernel,
        mesh=mesh,
        out_type=jax.ShapeDtypeStruct((n_rows // _K, _K * _CHUNK, D), jnp.float32),
        scratch_types=scratch,
        compiler_params=pltpu.CompilerParams(use_tc_tiling_on_sc=False),
    )
    def gather_kernel(table_hbm, idx_hbm, out_hbm, idx_v, *rest):
        rows = rest[:2]
        gsems = rest[2:4]
        wsems = rest[4:]
        wid = lax.axis_index("s") * _NC + lax.axis_index("c")
        base = wid * n_groups

        pltpu.sync_copy(idx_hbm.at[wid], idx_v)

        def start_gathers(g, h):
            for b in range(_K):
                j = g * _K + b
                pltpu.async_copy(
                    table_hbm.at[idx_v.at[j]],
                    rows[h].at[pl.ds(b * _CHUNK, _CHUNK)],
                    gsems[h],
                )

        def drain_gathers(h):
            for b in range(_K):
                pltpu.make_async_copy(
                    table_hbm.at[idx_v.at[0]],
                    rows[h].at[pl.ds(b * _CHUNK, _CHUNK)],
                    gsems[h],
                ).wait()

        def start_writebacks(g, h):
            pltpu.async_copy(rows[h], out_hbm.at[base + g], wsems[h])

        def drain_writebacks(h):
            pltpu.make_async_copy(rows[h], out_hbm.at[0], wsems[h]).wait()

        def half_step(g, h, o):
            drain_gathers(h)

            @pl.when(g + 1 < n_groups)
            def _():
                @pl.when(g >= 1)
                def _():
                    drain_writebacks(o)

                start_gathers(g + 1, o)

            start_writebacks(g, h)

        start_gathers(0, 0)

        def body(g, carry):
            half_step(g, 0, 1)
            half_step(g + 1, 1, 0)
            return carry

        lax.fori_loop(0, n_groups // 2, lambda i, c: body(i * 2, c), 0)
        drain_writebacks(1)

    return gather_kernel


def _seq_lens_body(x_ref, o_ref):
    o_ref[...] = jnp.sum((x_ref[...] != 0).astype(jnp.int32), axis=1)


def kernel(x, table):
    B_, L_ = x.shape
    V, D = table.shape
    N = B_ * L_
    x_i32 = x.astype(jnp.int32)
    idx3 = x_i32.reshape(_NW, N // (_NW * _CHUNK), _CHUNK)

    emb3 = _make_gather(V, D, N)(table, idx3)
    emb = emb3.reshape(B_, L_, D)

    seq_lens = pl.pallas_call(
        _seq_lens_body,
        out_shape=jax.ShapeDtypeStruct((B_,), jnp.int32),
    )(x_i32)

    return (emb, seq_lens)
